# Initial kernel scaffold; baseline (speedup 1.0000x reference)
import jax
import jax.numpy as jnp
from jax.experimental import pallas as pl


def kernel(input_ids, position_ids, var_idx, res_idx, leadtime_idx, spatial_temporal_features, token_table, var_table, res_table, leadtime_table, W1, b1, W2, b2):
    raise NotImplementedError("write your pallas kernel here")


# speedup vs baseline: 2.6515x; 2.6515x over previous
import functools

import jax
import jax.numpy as jnp
from jax import lax
from jax.experimental import pallas as pl
from jax.experimental.pallas import tpu as pltpu
from jax.experimental.pallas import tpu_sc as plsc


def _sc_gather(table, ids):
    n = ids.shape[0]
    h = table.shape[1]
    info = plsc.get_sparse_core_info()
    nw = info.num_cores * info.num_subcores
    per_w = n // nw
    ch = 32
    n_ch = per_w // ch
    mesh = plsc.VectorSubcoreMesh(core_axis_name="c", subcore_axis_name="s")

    @functools.partial(
        pl.kernel,
        out_type=jax.ShapeDtypeStruct((n, h), jnp.float32),
        mesh=mesh,
        scratch_types=[
            pltpu.VMEM((per_w,), jnp.int32),
            pltpu.VMEM((ch, h), jnp.float32),
            pltpu.VMEM((ch, h), jnp.float32),
            pltpu.SemaphoreType.DMA,
            pltpu.SemaphoreType.DMA,
        ],
    )
    def k(table_hbm, ids_hbm, out_hbm, idx_v, buf0, buf1, sem0, sem1):
        wid = lax.axis_index("s") * info.num_cores + lax.axis_index("c")
        base = wid * per_w
        pltpu.sync_copy(ids_hbm.at[pl.ds(base, per_w)], idx_v)
        bufs = (buf0, buf1)
        sems = (sem0, sem1)
        copies = [None, None]
        copies[0] = pltpu.async_copy(
            table_hbm.at[idx_v.at[pl.ds(0, ch)]], bufs[0], sems[0])
        for i in range(n_ch):
            s = i % 2
            copies[s].wait()
            if i + 1 < n_ch:
                copies[(i + 1) % 2] = pltpu.async_copy(
                    table_hbm.at[idx_v.at[pl.ds((i + 1) * ch, ch)]],
                    bufs[(i + 1) % 2], sems[(i + 1) % 2])
            pltpu.sync_copy(bufs[s], out_hbm.at[pl.ds(base + i * ch, ch)])

    return k(table, ids)


def _tc_combine(g, var_idx, res_idx, lt_idx, st8, var_t, res_t, lt_t,
                w1p, b1, w2, b2):
    n, h = g.shape
    tb = 512
    nblk = n // tb
    vv, vr, vl = var_t.shape[0], res_t.shape[0], lt_t.shape[0]
    std = w2.shape[0]

    vi3 = var_idx.reshape(nblk, 1, tb)
    ri3 = res_idx.reshape(nblk, 1, tb)
    li3 = lt_idx.reshape(nblk, 1, tb)

    def body(gref, vref, rref, lref, stref, vtref, rtref, ltref,
             w1ref, b1ref, w2ref, b2ref, oref):
        acc = gref[...]
        vi = vref[0, 0, :]
        ri = rref[0, 0, :]
        li = lref[0, 0, :]
        ohv = (vi[:, None] == lax.broadcasted_iota(jnp.int32, (tb, vv), 1)
               ).astype(jnp.bfloat16)
        ohr = (ri[:, None] == lax.broadcasted_iota(jnp.int32, (tb, vr), 1)
               ).astype(jnp.bfloat16)
        ohl = (li[:, None] == lax.broadcasted_iota(jnp.int32, (tb, vl), 1)
               ).astype(jnp.bfloat16)
        acc += jnp.dot(ohv, vtref[...].astype(jnp.bfloat16),
                       preferred_element_type=jnp.float32)
        acc += jnp.dot(ohr, rtref[...].astype(jnp.bfloat16),
                       preferred_element_type=jnp.float32)
        acc += jnp.dot(ohl, ltref[...].astype(jnp.bfloat16),
                       preferred_element_type=jnp.float32)
        hmid = jax.nn.gelu(
            jnp.dot(stref[...], w1ref[...], preferred_element_type=jnp.float32)
            + b1ref[0])
        acc += jnp.dot(hmid.astype(jnp.bfloat16),
                       w2ref[...].astype(jnp.bfloat16),
                       preferred_element_type=jnp.float32)
        acc += b2ref[0]
        oref[...] = acc

    return pl.pallas_call(
        body,
        grid=(nblk,),
        in_specs=[
            pl.BlockSpec((tb, h), lambda i: (i, 0)),
            pl.BlockSpec((1, 1, tb), lambda i: (i, 0, 0)),
            pl.BlockSpec((1, 1, tb), lambda i: (i, 0, 0)),
            pl.BlockSpec((1, 1, tb), lambda i: (i, 0, 0)),
            pl.BlockSpec((tb, 8), lambda i: (i, 0)),
            pl.BlockSpec((vv, h), lambda i: (0, 0)),
            pl.BlockSpec((vr, h), lambda i: (0, 0)),
            pl.BlockSpec((vl, h), lambda i: (0, 0)),
            pl.BlockSpec((8, std), lambda i: (0, 0)),
            pl.BlockSpec((1, std), lambda i: (0, 0)),
            pl.BlockSpec((std, h), lambda i: (0, 0)),
            pl.BlockSpec((1, h), lambda i: (0, 0)),
        ],
        out_specs=pl.BlockSpec((tb, h), lambda i: (i, 0)),
        out_shape=jax.ShapeDtypeStruct((n, h), jnp.float32),
    )(g, vi3, ri3, li3, st8, var_t, res_t, lt_t, w1p, b1, w2, b2)


def kernel(input_ids, position_ids, var_idx, res_idx, leadtime_idx,
           spatial_temporal_features, token_table, var_table, res_table,
           leadtime_table, W1, b1, W2, b2):
    n = input_ids.size
    ids = input_ids.reshape(n)
    g = _sc_gather(token_table, ids)
    st8 = jnp.pad(spatial_temporal_features.reshape(n, -1), ((0, 0), (0, 1)))
    w1p = jnp.pad(W1, ((0, 1), (0, 0)))
    out = _tc_combine(g, var_idx.reshape(n), res_idx.reshape(n),
                      leadtime_idx.reshape(n), st8, var_table, res_table,
                      leadtime_table, w1p, b1.reshape(1, -1), W2,
                      b2.reshape(1, -1))
    return (out, position_ids)


# speedup vs baseline: 3.0105x; 1.1354x over previous
import functools

import jax
import jax.numpy as jnp
from jax import lax
from jax.experimental import pallas as pl
from jax.experimental.pallas import tpu as pltpu
from jax.experimental.pallas import tpu_sc as plsc


def _sc_gather(table, ids):
    n = ids.shape[0]
    h = table.shape[1]
    info = plsc.get_sparse_core_info()
    nw = info.num_cores * info.num_subcores
    per_w = n // nw
    ch = 32
    n_ch = per_w // ch
    mesh = plsc.VectorSubcoreMesh(core_axis_name="c", subcore_axis_name="s")

    @functools.partial(
        pl.kernel,
        out_type=jax.ShapeDtypeStruct((n, h), jnp.float32),
        mesh=mesh,
        scratch_types=[
            pltpu.VMEM((per_w,), jnp.int32),
            pltpu.VMEM((ch, h), jnp.float32),
            pltpu.VMEM((ch, h), jnp.float32),
            pltpu.VMEM((ch, h), jnp.float32),
            pltpu.SemaphoreType.DMA,
            pltpu.SemaphoreType.DMA,
            pltpu.SemaphoreType.DMA,
            pltpu.SemaphoreType.DMA,
            pltpu.SemaphoreType.DMA,
            pltpu.SemaphoreType.DMA,
        ],
    )
    def k(table_hbm, ids_hbm, out_hbm, idx_v, buf0, buf1, buf2,
          gsem0, gsem1, gsem2, wsem0, wsem1, wsem2):
        wid = lax.axis_index("s") * info.num_cores + lax.axis_index("c")
        base = wid * per_w
        pltpu.sync_copy(ids_hbm.at[pl.ds(base, per_w)], idx_v)
        bufs = (buf0, buf1, buf2)
        gsems = (gsem0, gsem1, gsem2)
        wsems = (wsem0, wsem1, wsem2)
        nb = 3
        gathers = [None] * nb
        writes = [None] * nb
        for i in range(min(nb, n_ch)):
            gathers[i] = pltpu.async_copy(
                table_hbm.at[idx_v.at[pl.ds(i * ch, ch)]], bufs[i], gsems[i])
        for i in range(n_ch):
            s = i % nb
            gathers[s].wait()
            writes[s] = pltpu.async_copy(
                bufs[s], out_hbm.at[pl.ds(base + i * ch, ch)], wsems[s])
            if i + nb < n_ch:
                writes[s].wait()
                gathers[s] = pltpu.async_copy(
                    table_hbm.at[idx_v.at[pl.ds((i + nb) * ch, ch)]],
                    bufs[s], gsems[s])
        for i in range(max(n_ch - nb, 0), n_ch):
            writes[i % nb].wait()

    return k(table, ids)


def _tc_combine(g, var_idx, res_idx, lt_idx, st8, cat_t, w1p, b1, w2, b2):
    n, h = g.shape
    tb = 1024
    nblk = n // tb
    vc = cat_t.shape[0]
    std = w2.shape[0]

    vi3 = var_idx.reshape(nblk, 1, tb)
    ri3 = res_idx.reshape(nblk, 1, tb)
    li3 = lt_idx.reshape(nblk, 1, tb)

    def body(gref, vref, rref, lref, stref, ctref,
             w1ref, b1ref, w2ref, b2ref, oref):
        vi = vref[0, 0, :]
        ri = rref[0, 0, :] + 128
        li = lref[0, 0, :] + 144
        col = lax.broadcasted_iota(jnp.int32, (tb, vc), 1)
        mh = ((vi[:, None] == col) | (ri[:, None] == col) | (li[:, None] == col)
              ).astype(jnp.bfloat16)
        acc = gref[...]
        acc += jnp.dot(mh, ctref[...].astype(jnp.bfloat16),
                       preferred_element_type=jnp.float32)
        hmid = jax.nn.gelu(
            jnp.dot(stref[...], w1ref[...], preferred_element_type=jnp.float32)
            + b1ref[0])
        acc += jnp.dot(hmid.astype(jnp.bfloat16),
                       w2ref[...].astype(jnp.bfloat16),
                       preferred_element_type=jnp.float32)
        acc += b2ref[0]
        oref[...] = acc

    return pl.pallas_call(
        body,
        grid=(nblk,),
        in_specs=[
            pl.BlockSpec((tb, h), lambda i: (i, 0)),
            pl.BlockSpec((1, 1, tb), lambda i: (i, 0, 0)),
            pl.BlockSpec((1, 1, tb), lambda i: (i, 0, 0)),
            pl.BlockSpec((1, 1, tb), lambda i: (i, 0, 0)),
            pl.BlockSpec((tb, 8), lambda i: (i, 0)),
            pl.BlockSpec((vc, h), lambda i: (0, 0)),
            pl.BlockSpec((8, std), lambda i: (0, 0)),
            pl.BlockSpec((1, std), lambda i: (0, 0)),
            pl.BlockSpec((std, h), lambda i: (0, 0)),
            pl.BlockSpec((1, h), lambda i: (0, 0)),
        ],
        out_specs=pl.BlockSpec((tb, h), lambda i: (i, 0)),
        out_shape=jax.ShapeDtypeStruct((n, h), jnp.float32),
    )(g, vi3, ri3, li3, st8, cat_t, w1p, b1, w2, b2)


def kernel(input_ids, position_ids, var_idx, res_idx, leadtime_idx,
           spatial_temporal_features, token_table, var_table, res_table,
           leadtime_table, W1, b1, W2, b2):
    n = input_ids.size
    ids = input_ids.reshape(n)
    g = _sc_gather(token_table, ids)
    st8 = jnp.pad(spatial_temporal_features.reshape(n, -1), ((0, 0), (0, 1)))
    w1p = jnp.pad(W1, ((0, 1), (0, 0)))
    cat_t = jnp.concatenate([var_table, res_table, leadtime_table], axis=0)
    out = _tc_combine(g, var_idx.reshape(n), res_idx.reshape(n),
                      leadtime_idx.reshape(n), st8, cat_t,
                      w1p, b1.reshape(1, -1), W2, b2.reshape(1, -1))
    return (out, position_ids)
